# baseline (speedup 1.0000x reference)
import functools

import jax
import jax.numpy as jnp
from jax import lax
from jax.experimental import pallas as pl
from jax.experimental.pallas import tpu as pltpu
from jax.experimental.pallas import tpu_sc as plsc

L = 16


@functools.lru_cache(maxsize=None)
def _build_sc_pass(M, D, B):
    info = plsc.get_sparse_core_info()
    NC, NS = info.num_cores, info.num_subcores
    NW = NC * NS
    assert info.num_lanes == L and D == 64 and B % (4 * L) == 0
    assert B <= 1 << 14 and NW == 32
    CW = 512
    n_chunks = (M - M % (CW * NW)) // CW
    main = n_chunks * CW
    cpw = n_chunks // NW
    mesh = plsc.VectorSubcoreMesh(core_axis_name="c", subcore_axis_name="s")

    @functools.partial(
        pl.kernel,
        out_type=jax.ShapeDtypeStruct((D, M), jnp.float32),
        mesh=mesh,
        compiler_params=pltpu.CompilerParams(needs_layout_passes=False),
        scratch_types=[
            pltpu.VMEM((B + L,), jnp.int32),
            pltpu.VMEM((B + L,), jnp.int32),
            pltpu.VMEM((L,), jnp.int32),
            pltpu.VMEM((D, CW), jnp.float32),
            pltpu.VMEM((D, CW), jnp.float32),
            pltpu.VMEM((L, 2 * D), jnp.float32),
            pltpu.SMEM((1,), jnp.int32),
            pltpu.SemaphoreType.DMA,
            pltpu.SemaphoreType.DMA,
            pltpu.SemaphoreType.DMA,
            pltpu.SemaphoreType.DMA,
            pltpu.SemaphoreType.DMA,
        ],
    )
    def sc_pass(xt_hbm, y_hbm, zp_hbm, acc_hbm, out_hbm,
                cl, own_l, acc_v, cb0, cb1, zbuf, kc_ref,
                si0, si1, so0, so1, sem_z):
        wid = lax.axis_index("c") * NS + lax.axis_index("s")
        y_v = cl

        pltpu.sync_copy(y_hbm, y_v.at[pl.ds(0, B)])
        pltpu.sync_copy(acc_hbm, acc_v)
        acc_s = jnp.max(acc_v[...])
        lane = lax.iota(jnp.int32, L)

        def scan_body(q, cnt):
            for u in range(4):
                v = q * 4 + u
                yv = y_v[pl.ds(v * L, L)]
                owner = lax.shift_right_logical(yv, 9) & (NW - 1)
                m = (yv < main) & (owner == wid)
                packed = (
                    lax.shift_left(lax.shift_right_logical(yv, 14), 23)
                    | lax.shift_left(yv & (CW - 1), 14)
                    | (lane + v * L)
                )
                csum = plsc.cumsum(m.astype(jnp.int32))
                plsc.store_scatter(own_l, [cnt + csum - 1], packed, mask=m)
                cnt = cnt + jnp.max(csum)
            return cnt

        cnt = lax.fori_loop(0, B // L // 4, scan_body, jnp.int32(0))
        n_own_vec = (cnt + L - 1) // L

        def compact(t):
            def compact_body(v, kc):
                pk = own_l[pl.ds(v * L, L)]
                m = ((lane + v * L) < cnt) & (
                    lax.shift_right_logical(pk, 23) == t)
                csum = plsc.cumsum(m.astype(jnp.int32))
                plsc.store_scatter(cl, [kc + csum - 1], pk, mask=m)
                return kc + jnp.max(csum)

            return lax.fori_loop(0, n_own_vec, compact_body, jnp.int32(0))

        def batches(t, kc, cb):
            def batch_body(b, _):
                base = b * L
                pk = cl[pl.ds(base, L)]
                nvalid = kc - base
                valid = lane < nvalid
                key = lax.shift_right_logical(pk, 14)
                col_off = jnp.where(valid, key & (CW - 1), 0)
                posv = jnp.where(valid, pk & (B - 1), 0)

                gz = pltpu.async_copy(zp_hbm.at[posv], zbuf, sem_z)

                @pl.when(acc_s != 0)
                def _():
                    rank = jnp.zeros((L,), jnp.int32)
                    for k in range(1, L):
                        up = lane - k
                        upw = jnp.where(up < 0, up + L, up)
                        rot = jnp.take_along_axis(key, upw, axis=0)
                        rank = rank + ((rot == key) & (up >= 0))
                    maxrank = jnp.max(jnp.where(valid, rank, 0))
                    gz.wait()

                    def round_body(r, _):
                        sel = valid & (rank == r)
                        for c in range(D):
                            cvec = jnp.full((L,), c, jnp.int32)
                            vals = plsc.load_gather(zbuf, [lane, cvec])
                            plsc.addupdate_scatter(cb, [cvec, col_off], vals,
                                                   mask=sel)
                        return 0

                    lax.fori_loop(0, maxrank + 1, round_body, 0)

                @pl.when(acc_s == 0)
                def _():
                    lo = lane
                    for k in range(1, L):
                        dn = lane + k
                        dnw = jnp.where(dn >= L, dn - L, dn)
                        rot = jnp.take_along_axis(key, dnw, axis=0)
                        eqd = (rot == key) & (dn < jnp.minimum(L, nvalid))
                        lo = jnp.where(eqd, jnp.maximum(lo, dnw), lo)
                    sel = valid & (lo == lane)
                    gz.wait()
                    for c in range(D):
                        cvec = jnp.full((L,), c, jnp.int32)
                        vals = plsc.load_gather(zbuf, [lane, cvec])
                        plsc.store_scatter(cb, [cvec, col_off], vals,
                                           mask=sel)
                return 0

            lax.fori_loop(0, (kc + L - 1) // L, batch_body, 0)

        bufs = ((cb0, si0, so0), (cb1, si1, so1))

        def col0_of(t):
            return (wid + NW * t) * CW

        def stage(t, phase):
            cb, si, _ = bufs[phase]
            ocb, osi, oso = bufs[1 - phase]
            del oso
            _, _, so = bufs[phase]
            t_next = jnp.minimum(t + 1, cpw - 1)

            @pl.when(t < cpw)
            def _():
                gin = pltpu.async_copy(
                    xt_hbm.at[:, pl.ds(col0_of(t_next), CW)], ocb, osi)
                batches(t, kc_ref[0], cb)
                gout = pltpu.async_copy(
                    cb, out_hbm.at[:, pl.ds(col0_of(t), CW)], so)
                kc_ref[0] = compact(t_next)
                gin.wait()
                gout.wait()

        gin0 = pltpu.async_copy(xt_hbm.at[:, pl.ds(col0_of(0), CW)], cb0, si0)
        kc_ref[0] = compact(jnp.int32(0))
        gin0.wait()

        def pair(p, _):
            stage(2 * p, 0)
            stage(2 * p + 1, 1)
            return 0

        lax.fori_loop(0, (cpw + 2) // 2, pair, 0)

    return sc_pass, main


def kernel(x, y, z, acc):
    M, D = x.shape
    B = y.shape[0]
    xt = x.T
    z_pad = jnp.pad(z, ((0, 0), (0, D)))
    acc_v = jnp.full((L,), 0, jnp.int32) + jnp.asarray(acc, jnp.int32)
    sc_pass, main = _build_sc_pass(M, D, B)
    out_t = sc_pass(xt, y, z_pad, acc_v)

    tail_x = lax.slice(x, (main, 0), (M, D))
    y_t = y - main

    def _tail_add(o):
        tx, yt, zz = o
        onehot = (yt[None, :] == jnp.arange(M - main, dtype=yt.dtype)[:, None])
        return tx + jnp.matmul(onehot.astype(zz.dtype), zz,
                               precision=lax.Precision.HIGHEST)

    tail_res = lax.cond(
        acc,
        _tail_add,
        lambda o: o[0].at[o[1]].set(o[2], mode="drop"),
        (tail_x, y_t, z),
    )
    out_t = lax.dynamic_update_slice(out_t, tail_res.T, (0, main))
    return out_t.T


# speedup vs baseline: 1.0015x; 1.0015x over previous
import functools

import jax
import jax.numpy as jnp
from jax import lax
from jax.experimental import pallas as pl
from jax.experimental.pallas import tpu as pltpu
from jax.experimental.pallas import tpu_sc as plsc

L = 16


@functools.lru_cache(maxsize=None)
def _build_sc_pass(M, D, B):
    info = plsc.get_sparse_core_info()
    NC, NS = info.num_cores, info.num_subcores
    NW = NC * NS
    assert info.num_lanes == L and D == 64 and B % (4 * L) == 0
    assert B <= 1 << 14 and NW == 32
    CW = 512
    n_chunks = (M - M % (CW * NW)) // CW
    main = n_chunks * CW
    cpw = n_chunks // NW
    mesh = plsc.VectorSubcoreMesh(core_axis_name="c", subcore_axis_name="s")

    @functools.partial(
        pl.kernel,
        out_type=jax.ShapeDtypeStruct((D, M), jnp.float32),
        mesh=mesh,
        compiler_params=pltpu.CompilerParams(needs_layout_passes=False),
        scratch_types=[
            pltpu.VMEM((B + L,), jnp.int32),
            pltpu.VMEM((B + L,), jnp.int32),
            pltpu.VMEM((L,), jnp.int32),
            pltpu.VMEM((D, CW), jnp.float32),
            pltpu.VMEM((D, CW), jnp.float32),
            pltpu.VMEM((L, 2 * D), jnp.float32),
            pltpu.SMEM((1,), jnp.int32),
            pltpu.SemaphoreType.DMA,
            pltpu.SemaphoreType.DMA,
            pltpu.SemaphoreType.DMA,
            pltpu.SemaphoreType.DMA,
            pltpu.SemaphoreType.DMA,
        ],
    )
    def sc_pass(xt_hbm, y_hbm, zp_hbm, acc_hbm, out_hbm,
                cl, own_l, acc_v, cb0, cb1, zbuf, kc_ref,
                si0, si1, so0, so1, sem_z):
        wid = lax.axis_index("c") * NS + lax.axis_index("s")
        y_v = cl

        pltpu.sync_copy(y_hbm, y_v.at[pl.ds(0, B)])
        pltpu.sync_copy(acc_hbm, acc_v)
        acc_s = jnp.max(acc_v[...])
        lane = lax.iota(jnp.int32, L)

        def scan_body(q, cnt_v):
            for u in range(4):
                v = q * 4 + u
                yv = y_v[pl.ds(v * L, L)]
                owner = lax.shift_right_logical(yv, 9) & (NW - 1)
                m = (yv < main) & (owner == wid)
                packed = (
                    lax.shift_left(lax.shift_right_logical(yv, 14), 23)
                    | lax.shift_left(yv & (CW - 1), 14)
                    | (lane + v * L)
                )
                csum = plsc.cumsum(m.astype(jnp.int32))
                plsc.store_scatter(own_l, [cnt_v + csum - 1], packed, mask=m)
                cnt_v = cnt_v + plsc.all_reduce_population_count(m)
            return cnt_v

        cnt = jnp.max(lax.fori_loop(0, B // L // 4, scan_body,
                                    jnp.zeros((L,), jnp.int32)))
        n_own_vec = (cnt + L - 1) // L

        def compact(t):
            def compact_body(v, kc_v):
                pk = own_l[pl.ds(v * L, L)]
                m = ((lane + v * L) < cnt) & (
                    lax.shift_right_logical(pk, 23) == t)
                csum = plsc.cumsum(m.astype(jnp.int32))
                plsc.store_scatter(cl, [kc_v + csum - 1], pk, mask=m)
                return kc_v + plsc.all_reduce_population_count(m)

            return jnp.max(lax.fori_loop(0, n_own_vec, compact_body,
                                         jnp.zeros((L,), jnp.int32)))

        def batches(t, kc, cb):
            def batch_body(b, _):
                base = b * L
                pk = cl[pl.ds(base, L)]
                nvalid = kc - base
                valid = lane < nvalid
                key = lax.shift_right_logical(pk, 14)
                col_off = jnp.where(valid, key & (CW - 1), 0)
                posv = jnp.where(valid, pk & (B - 1), 0)

                gz = pltpu.async_copy(zp_hbm.at[posv], zbuf, sem_z)

                @pl.when(acc_s != 0)
                def _():
                    rank = jnp.zeros((L,), jnp.int32)
                    for k in range(1, L):
                        up = lane - k
                        upw = jnp.where(up < 0, up + L, up)
                        rot = jnp.take_along_axis(key, upw, axis=0)
                        rank = rank + ((rot == key) & (up >= 0))
                    maxrank = jnp.max(jnp.where(valid, rank, 0))
                    gz.wait()

                    def round_body(r, _):
                        sel = valid & (rank == r)
                        for c in range(D):
                            cvec = jnp.full((L,), c, jnp.int32)
                            vals = plsc.load_gather(zbuf, [lane, cvec])
                            plsc.addupdate_scatter(cb, [cvec, col_off], vals,
                                                   mask=sel)
                        return 0

                    lax.fori_loop(0, maxrank + 1, round_body, 0)

                @pl.when(acc_s == 0)
                def _():
                    lo = lane
                    for k in range(1, L):
                        dn = lane + k
                        dnw = jnp.where(dn >= L, dn - L, dn)
                        rot = jnp.take_along_axis(key, dnw, axis=0)
                        eqd = (rot == key) & (dn < jnp.minimum(L, nvalid))
                        lo = jnp.where(eqd, jnp.maximum(lo, dnw), lo)
                    sel = valid & (lo == lane)
                    gz.wait()
                    for c in range(D):
                        cvec = jnp.full((L,), c, jnp.int32)
                        vals = plsc.load_gather(zbuf, [lane, cvec])
                        plsc.store_scatter(cb, [cvec, col_off], vals,
                                           mask=sel)
                return 0

            lax.fori_loop(0, (kc + L - 1) // L, batch_body, 0)

        bufs = ((cb0, si0, so0), (cb1, si1, so1))

        def col0_of(t):
            return (wid + NW * t) * CW

        def stage(t, phase):
            cb, si, _ = bufs[phase]
            ocb, osi, oso = bufs[1 - phase]
            del oso
            _, _, so = bufs[phase]
            t_next = jnp.minimum(t + 1, cpw - 1)

            @pl.when(t < cpw)
            def _():
                gin = pltpu.async_copy(
                    xt_hbm.at[:, pl.ds(col0_of(t_next), CW)], ocb, osi)
                batches(t, kc_ref[0], cb)
                gout = pltpu.async_copy(
                    cb, out_hbm.at[:, pl.ds(col0_of(t), CW)], so)
                kc_ref[0] = compact(t_next)
                gin.wait()
                gout.wait()

        gin0 = pltpu.async_copy(xt_hbm.at[:, pl.ds(col0_of(0), CW)], cb0, si0)
        kc_ref[0] = compact(jnp.int32(0))
        gin0.wait()

        def pair(p, _):
            stage(2 * p, 0)
            stage(2 * p + 1, 1)
            return 0

        lax.fori_loop(0, (cpw + 2) // 2, pair, 0)

    return sc_pass, main


def kernel(x, y, z, acc):
    M, D = x.shape
    B = y.shape[0]
    xt = x.T
    z_pad = jnp.pad(z, ((0, 0), (0, D)))
    acc_v = jnp.full((L,), 0, jnp.int32) + jnp.asarray(acc, jnp.int32)
    sc_pass, main = _build_sc_pass(M, D, B)
    out_t = sc_pass(xt, y, z_pad, acc_v)

    tail_x = lax.slice(x, (main, 0), (M, D))
    y_t = y - main

    def _tail_add(o):
        tx, yt, zz = o
        onehot = (yt[None, :] == jnp.arange(M - main, dtype=yt.dtype)[:, None])
        return tx + jnp.matmul(onehot.astype(zz.dtype), zz,
                               precision=lax.Precision.HIGHEST)

    tail_res = lax.cond(
        acc,
        _tail_add,
        lambda o: o[0].at[o[1]].set(o[2], mode="drop"),
        (tail_x, y_t, z),
    )
    out_t = lax.dynamic_update_slice(out_t, tail_res.T, (0, main))
    return out_t.T
